# baseline (speedup 1.0000x reference)
import jax
import jax.numpy as jnp
from jax import lax
from jax.experimental import pallas as pl
from jax.experimental.pallas import tpu as pltpu
from jax.experimental.pallas import tpu_sc as plsc

B = 1024
T = 50
NF = 26
FIELD_V = 38461
EMB = 16
NORM_DIM = NF * EMB
EPS = 1e-5

NTOK = B * T
NW = 32
CHUNKS_W = B // NW
ROWS_PER_CHUNK = T * NF
IDX_PAD = 1304
GATHER_OFFS = [104 * j for j in range(13)]
GATHER_SIZES = [104] * 12 + [52]
ROWS_PAD = 64 * NF


def _sc_body(idx_hbm, table_hbm, gamma_hbm, beta_hbm, out_hbm,
             idx_v, rows_v, stage_v, gamma_v, beta_v, gsems, osems):
    wid = lax.axis_index("s") * 2 + lax.axis_index("c")
    pltpu.sync_copy(gamma_hbm, gamma_v)
    pltpu.sync_copy(beta_hbm, beta_v)
    chunk0 = wid * CHUNKS_W

    lane = jnp.arange(16, dtype=jnp.int32)
    skew = [jnp.bitwise_and(lane + u, 15) for u in range(16)]

    def fire_chunk(k, b):
        pltpu.sync_copy(
            idx_hbm.at[pl.ds((chunk0 + k) * IDX_PAD, ROWS_PER_CHUNK)],
            idx_v.at[b])
        for off, sz in zip(GATHER_OFFS, GATHER_SIZES):
            pltpu.make_async_copy(
                table_hbm.at[idx_v.at[b].at[pl.ds(off, sz)]],
                rows_v.at[b].at[pl.ds(off, sz)],
                gsems.at[b],
            ).start()

    def wait_chunk(b):
        for off, sz in zip(GATHER_OFFS, GATHER_SIZES):
            pltpu.make_async_copy(
                table_hbm.at[idx_v.at[b].at[pl.ds(off, sz)]],
                rows_v.at[b].at[pl.ds(off, sz)],
                gsems.at[b],
            ).wait()

    def out_copy(k, b):
        return pltpu.make_async_copy(
            stage_v.at[b], out_hbm.at[chunk0 + k], osems.at[b])

    def compute_chunk(b):
        rows_b = rows_v.at[b]
        stage_b = stage_v.at[b]

        for g in range(4):
            t_vec = lane + g * 16
            if g < 3:
                tok = t_vec
                mask = None
            else:
                mask = lane < (T - 48)
                tok = jnp.where(mask, t_vec, 0)
            base_row = t_vec * NF

            def pass1(f, acc):
                s1a, s1b, s1c, s1d, s2a, s2b, s2c, s2d = acc
                r = base_row + f
                s1 = [s1a, s1b, s1c, s1d]
                s2 = [s2a, s2b, s2c, s2d]
                for u in range(16):
                    v = plsc.load_gather(rows_b, [r, skew[u]])
                    s1[u % 4] = s1[u % 4] + v
                    s2[u % 4] = s2[u % 4] + v * v
                return (*s1, *s2)

            zero = jnp.zeros((16,), jnp.float32)
            accs = lax.fori_loop(0, NF, pass1, (zero,) * 8)
            s1 = (accs[0] + accs[1]) + (accs[2] + accs[3])
            s2 = (accs[4] + accs[5]) + (accs[6] + accs[7])
            mean = s1 * (1.0 / NORM_DIM)
            var = s2 * (1.0 / NORM_DIM) - mean * mean
            x = var + EPS
            i = jnp.int32(0x5F3759DF) - lax.shift_right_arithmetic(
                plsc.bitcast(x, jnp.int32), 1)
            y = plsc.bitcast(i, jnp.float32)
            for _ in range(3):
                y = y * (1.5 - 0.5 * x * y * y)
            rstd = y

            def pass2(f, _):
                r = base_row + f
                fe = f * 16
                gvec = gamma_v[pl.ds(fe, 16)]
                bvec = beta_v[pl.ds(fe, 16)]
                for u in range(16):
                    v = plsc.load_gather(rows_b, [r, skew[u]])
                    gb = gvec.at[skew[u]].get(mode="promise_in_bounds")
                    bb = bvec.at[skew[u]].get(mode="promise_in_bounds")
                    o = (v - mean) * rstd
                    o = o * gb + bb
                    plsc.store_scatter(stage_b, [tok, fe + skew[u]], o,
                                       mask=mask)
                return 0

            lax.fori_loop(0, NF, pass2, 0)

    fire_chunk(0, 0)

    def chunk_body(k, carry):
        b = lax.rem(k, 2)

        @pl.when(k >= 2)
        def _():
            out_copy(k - 2, b).wait()

        @pl.when(k + 1 < CHUNKS_W)
        def _():
            fire_chunk(k + 1, 1 - b)

        wait_chunk(b)
        compute_chunk(b)
        out_copy(k, b).start()
        return carry

    lax.fori_loop(0, CHUNKS_W, chunk_body, 0)
    out_copy(CHUNKS_W - 2, lax.rem(CHUNKS_W - 2, 2)).wait()
    out_copy(CHUNKS_W - 1, lax.rem(CHUNKS_W - 1, 2)).wait()


@jax.jit
def _sc_call(idx_padded, table, gamma, beta):
    mesh = plsc.VectorSubcoreMesh(core_axis_name="c", subcore_axis_name="s")
    f = pl.kernel(
        _sc_body,
        out_type=jax.ShapeDtypeStruct((B, T, NORM_DIM), jnp.float32),
        mesh=mesh,
        scratch_types=[
            pltpu.VMEM((2, ROWS_PER_CHUNK), jnp.int32),
            pltpu.VMEM((2, ROWS_PAD, EMB), jnp.float32),
            pltpu.VMEM((2, T, NORM_DIM), jnp.float32),
            pltpu.VMEM((NORM_DIM,), jnp.float32),
            pltpu.VMEM((NORM_DIM,), jnp.float32),
            pltpu.SemaphoreType.DMA((2,)),
            pltpu.SemaphoreType.DMA((2,)),
        ],
        compiler_params=pltpu.CompilerParams(
            needs_layout_passes=False, use_tc_tiling_on_sc=False),
    )
    return f(idx_padded, table, gamma, beta)


def kernel(cate_x, mask, table, gamma, beta):
    offsets = jnp.arange(NF, dtype=cate_x.dtype) * FIELD_V
    shifted = cate_x + mask[:, :, None] * offsets[None, None, :]
    idx_padded = jnp.pad(
        shifted.reshape(B, T * NF), ((0, 0), (0, IDX_PAD - ROWS_PER_CHUNK))
    ).reshape(B * IDX_PAD)
    return _sc_call(idx_padded, table, gamma, beta)


# speedup vs baseline: 1.3353x; 1.3353x over previous
import jax
import jax.numpy as jnp
from jax import lax
from jax.experimental import pallas as pl
from jax.experimental.pallas import tpu as pltpu
from jax.experimental.pallas import tpu_sc as plsc

B = 1024
T = 50
NF = 26
FIELD_V = 38461
EMB = 16
NORM_DIM = NF * EMB
EPS = 1e-5

NW = 32
BATCHES_W = B // NW
PAIRS_W = BATCHES_W // 2
BR = T * NF
PR = 2 * BR
NGD = 25
GSZ = PR // NGD


def _sc_body(idx_hbm, table_hbm, gamma_hbm, beta_hbm, out_hbm,
             idx_v, rows_v, stage_v, gamma_v, beta_v, gsems, osems, isems):
    wid = lax.axis_index("s") * 2 + lax.axis_index("c")
    pltpu.sync_copy(gamma_hbm, gamma_v)
    pltpu.sync_copy(beta_hbm, beta_v)
    batch0 = wid * BATCHES_W
    pair0 = wid * PAIRS_W

    lane = jnp.arange(16, dtype=jnp.int32)
    skew = [jnp.bitwise_and(lane + u, 15) for u in range(16)]

    def idx_cp(p, s):
        return pltpu.make_async_copy(
            idx_hbm.at[pl.ds((pair0 + p) * PR, PR)], idx_v.at[s], isems.at[s])

    def gather_cp(j, b, s):
        return pltpu.make_async_copy(
            table_hbm.at[idx_v.at[s].at[pl.ds(j * GSZ, GSZ)]],
            rows_v.at[b].at[pl.ds(j * GSZ, GSZ)],
            gsems.at[b])

    def out_cp(k):
        return pltpu.make_async_copy(
            stage_v, out_hbm.at[batch0 + k], osems)

    def compute(b, half):
        rows_b = rows_v.at[b]
        base = half * BR

        for g in range(4):
            msk = (lane < (T - 48)) if g == 3 else None
            t_vec = lane + g * 16
            base_row = t_vec * NF + base

            def pass1(f, acc):
                s1a, s1b, s1c, s1d, s2a, s2b, s2c, s2d = acc
                r = base_row + f
                s1 = [s1a, s1b, s1c, s1d]
                s2 = [s2a, s2b, s2c, s2d]
                for u in range(16):
                    v = plsc.load_gather(rows_b, [r, skew[u]], mask=msk)
                    s1[u % 4] = s1[u % 4] + v
                    s2[u % 4] = s2[u % 4] + v * v
                return (*s1, *s2)

            zero = jnp.zeros((16,), jnp.float32)
            accs = lax.fori_loop(0, NF, pass1, (zero,) * 8)
            s1 = (accs[0] + accs[1]) + (accs[2] + accs[3])
            s2 = (accs[4] + accs[5]) + (accs[6] + accs[7])
            mean = s1 * (1.0 / NORM_DIM)
            var = s2 * (1.0 / NORM_DIM) - mean * mean
            x = var + EPS
            i = jnp.int32(0x5F3759DF) - lax.shift_right_arithmetic(
                plsc.bitcast(x, jnp.int32), 1)
            y = plsc.bitcast(i, jnp.float32)
            for _ in range(3):
                y = y * (1.5 - 0.5 * x * y * y)
            rstd = y

            def pass2(f, _):
                r = base_row + f
                fe = f * 16
                gvec = gamma_v[pl.ds(fe, 16)]
                bvec = beta_v[pl.ds(fe, 16)]
                vs = [plsc.load_gather(rows_b, [r, skew[u]], mask=msk)
                      for u in range(16)]
                for u in range(16):
                    gb = gvec.at[skew[u]].get(mode="promise_in_bounds")
                    bb = bvec.at[skew[u]].get(mode="promise_in_bounds")
                    a = rstd * gb
                    c = bb - mean * a
                    o = vs[u] * a + c
                    plsc.store_scatter(stage_v, [t_vec, fe + skew[u]], o,
                                       mask=msk)
                return 0

            lax.fori_loop(0, NF, pass2, 0)

    idx_cp(0, 0).start()
    idx_cp(0, 0).wait()
    for j in range(NGD):
        gather_cp(j, 0, 0).start()
    idx_cp(1, 1).start()

    def pair_body(p, carry):
        b = lax.rem(p, 2)
        s_cur = lax.rem(p, 3)
        s_nxt = lax.rem(p + 1, 3)
        s_n2 = lax.rem(p + 2, 3)

        @pl.when(p + 1 < PAIRS_W)
        def _():
            idx_cp(p + 1, s_nxt).wait()
            for j in range(NGD):
                gather_cp(j, 1 - b, s_nxt).start()

            @pl.when(p + 2 < PAIRS_W)
            def _():
                idx_cp(p + 2, s_n2).start()

        for j in range(NGD):
            gather_cp(j, b, s_cur).wait()

        @pl.when(p >= 1)
        def _():
            out_cp(2 * p - 1).wait()

        compute(b, 0)
        out_cp(2 * p).start()
        out_cp(2 * p).wait()
        compute(b, 1)
        out_cp(2 * p + 1).start()
        return carry

    lax.fori_loop(0, PAIRS_W, pair_body, 0)
    out_cp(2 * PAIRS_W - 1).wait()


@jax.jit
def _sc_call(idx_flat, table, gamma, beta):
    mesh = plsc.VectorSubcoreMesh(core_axis_name="c", subcore_axis_name="s")
    f = pl.kernel(
        _sc_body,
        out_type=jax.ShapeDtypeStruct((B, T, NORM_DIM), jnp.float32),
        mesh=mesh,
        scratch_types=[
            pltpu.VMEM((3, PR), jnp.int32),
            pltpu.VMEM((2, PR, EMB), jnp.float32),
            pltpu.VMEM((T, NORM_DIM), jnp.float32),
            pltpu.VMEM((NORM_DIM,), jnp.float32),
            pltpu.VMEM((NORM_DIM,), jnp.float32),
            pltpu.SemaphoreType.DMA((2,)),
            pltpu.SemaphoreType.DMA,
            pltpu.SemaphoreType.DMA((3,)),
        ],
        compiler_params=pltpu.CompilerParams(
            needs_layout_passes=False, use_tc_tiling_on_sc=False),
    )
    return f(idx_flat, table, gamma, beta)


def kernel(cate_x, mask, table, gamma, beta):
    offsets = jnp.arange(NF, dtype=cate_x.dtype) * FIELD_V
    shifted = cate_x + mask[:, :, None] * offsets[None, None, :]
    return _sc_call(shifted.reshape(B * T * NF), table, gamma, beta)
